# baseline (speedup 1.0000x reference)
import functools

import jax
import jax.numpy as jnp
from jax import lax
from jax.experimental import pallas as pl
from jax.experimental.pallas import tpu as pltpu
from jax.experimental.pallas import tpu_sc as plsc

N = 10000
E = 320000
D = 128
DC = 64

NC = 2
NS = 16
K = 128
C = 160
PHASES = 10
CP = C // PHASES
NSLOT = 2
T = CP // NSLOT
EPT = K * C
E_PAD = NS * EPT
NPAD = 10112
RPT = NPAD // NS
DUMMY = N + 8
_RCHUNKS = [128, 128, 128, 128, 120]
_XROWS = 624
_XCHUNKS = [128, 128, 128, 128, 112]

_mesh = plsc.VectorSubcoreMesh(core_axis_name="c", subcore_axis_name="s")


@functools.partial(
    pl.kernel,
    out_type=jax.ShapeDtypeStruct((NC, NS, RPT, DC), jnp.float32),
    mesh=_mesh,
    scratch_types=[
        pltpu.VMEM_SHARED((N, DC), jnp.float32),
        pltpu.VMEM_SHARED((NPAD, DC), jnp.float32),
        pltpu.VMEM((CP * K,), jnp.int32),
        [pltpu.VMEM((1, K), jnp.int32) for _ in range(NSLOT)],
        [pltpu.VMEM((K, DC), jnp.float32) for _ in range(NSLOT)],
        [pltpu.SemaphoreType.DMA for _ in range(3 * NSLOT)],
    ],
)
def _segment_sum_sc(xt_hbm, src_hbm, dst_hbm, zero_hbm, out_hbm,
                    xsp, acc, src_v, dbufs, bufs, sems):
    gsems = sems[0:NSLOT]
    dsems = sems[NSLOT:2 * NSLOT]
    ssems = sems[2 * NSLOT:3 * NSLOT]
    c = lax.axis_index("c")
    s = lax.axis_index("s")

    xb = s * _XROWS
    off = 0
    for i, sz in enumerate(_XCHUNKS):
        b = bufs[i % NSLOT]
        pltpu.sync_copy(xt_hbm.at[c, pl.ds(xb + off, sz)], b.at[pl.ds(0, sz)])
        pltpu.sync_copy(b.at[pl.ds(0, sz)], xsp.at[pl.ds(xb + off, sz)])
        off += sz

    @pl.when(s == 0)
    def _():
        xtail = NS * _XROWS
        pltpu.sync_copy(xt_hbm.at[c, pl.ds(xtail, N - xtail)],
                        bufs[0].at[pl.ds(0, N - xtail)])
        pltpu.sync_copy(bufs[0].at[pl.ds(0, N - xtail)],
                        xsp.at[pl.ds(xtail, N - xtail)])

    r0 = s * RPT
    pltpu.sync_copy(zero_hbm, bufs[1])
    off = 0
    for i, sz in enumerate(_RCHUNKS):
        pltpu.sync_copy(bufs[1].at[pl.ds(0, sz)], acc.at[pl.ds(r0 + off, sz)])
        off += sz

    plsc.subcore_barrier()

    def start_in(pbase, j, i):
        pltpu.async_copy(xsp.at[src_v.at[pl.ds(j * K, K)]], bufs[i], gsems[i])
        pltpu.async_copy(dst_hbm.at[pl.ds(s * C + pbase + j, 1)],
                         dbufs[i], dsems[i])

    def wait_in(i):
        pltpu.make_async_copy(xsp.at[src_v.at[pl.ds(0, K)]],
                              bufs[i], gsems[i]).wait()
        pltpu.make_async_copy(dst_hbm.at[pl.ds(0, 1)], dbufs[i],
                              dsems[i]).wait()

    def start_scat(i):
        pltpu.async_copy(bufs[i], acc.at[dbufs[i].at[0]], ssems[i], add=True)

    def wait_scat(i):
        pltpu.make_async_copy(bufs[i], acc.at[dbufs[i].at[0]],
                              ssems[i]).wait()

    for p in range(PHASES):
        pbase = p * CP
        pltpu.sync_copy(src_hbm.at[pl.ds(s * EPT + pbase * K, CP * K)], src_v)
        for i in range(NSLOT):
            start_in(pbase, i, i)

        def body(t, carry, pbase=pbase):
            j0 = NSLOT * t
            for i in range(NSLOT):
                wait_in(i)
                start_scat(i)
            for i in range(NSLOT):
                wait_scat(i)

                @pl.when(t + 1 < T)
                def _(i=i):
                    start_in(pbase, j0 + NSLOT + i, i)
            return carry

        lax.fori_loop(0, T, body, 0)
    plsc.subcore_barrier()

    off = 0
    for i, sz in enumerate(_RCHUNKS):
        b = bufs[i % NSLOT]
        pltpu.sync_copy(acc.at[pl.ds(r0 + off, sz)], b.at[pl.ds(0, sz)])
        pltpu.sync_copy(b.at[pl.ds(0, sz)], out_hbm.at[c, s, pl.ds(off, sz)])
        off += sz


def _mlp_body(x_ref, p0_ref, p1_ref, w1_ref, b1_ref, w2_ref, b2_ref, o_ref):
    agg = jnp.concatenate([p0_ref[0], p1_ref[0]], axis=1)
    sm = x_ref[...] + agg
    h = jnp.dot(sm, w1_ref[...], preferred_element_type=jnp.float32)
    h = jnp.maximum(h + b1_ref[...], 0.0)
    o = jnp.dot(h, w2_ref[...], preferred_element_type=jnp.float32)
    o_ref[...] = jnp.maximum(o + b2_ref[...], 0.0)


_BLK = 1000


def _mlp_tc(x, parts, W1, b1, W2, b2):
    grid = (N // _BLK,)
    return pl.pallas_call(
        _mlp_body,
        grid=grid,
        in_specs=[
            pl.BlockSpec((_BLK, D), lambda i: (i, 0)),
            pl.BlockSpec((1, _BLK, DC), lambda i: (0, i, 0)),
            pl.BlockSpec((1, _BLK, DC), lambda i: (1, i, 0)),
            pl.BlockSpec((D, D), lambda i: (0, 0)),
            pl.BlockSpec((1, D), lambda i: (0, 0)),
            pl.BlockSpec((D, D), lambda i: (0, 0)),
            pl.BlockSpec((1, D), lambda i: (0, 0)),
        ],
        out_specs=pl.BlockSpec((_BLK, D), lambda i: (i, 0)),
        out_shape=jax.ShapeDtypeStruct((N, D), jnp.float32),
    )(x, parts, parts, W1, b1, W2, b2)


def kernel(x, edge_index, W1a, b1a, W1b, b1b, W2a, b2a, W2b, b2b):
    pad = E_PAD - E
    src = jnp.concatenate([edge_index[0], jnp.zeros((pad,), jnp.int32)])
    dst = jnp.concatenate([edge_index[1], jnp.full((pad,), DUMMY, jnp.int32)])
    dst = dst.reshape(NS * C, K)
    zero = jnp.zeros((K, DC), jnp.float32)

    xt = jnp.stack([x[:, :DC], x[:, DC:]], axis=0)
    parts1 = _segment_sum_sc(xt, src, dst, zero).reshape(NC, NPAD, DC)
    h1 = _mlp_tc(x, parts1, W1a, b1a.reshape(1, D), W1b, b1b.reshape(1, D))
    h1t = jnp.stack([h1[:, :DC], h1[:, DC:]], axis=0)
    parts2 = _segment_sum_sc(h1t, src, dst, zero).reshape(NC, NPAD, DC)
    h2 = _mlp_tc(h1, parts2, W2a, b2a.reshape(1, D), W2b, b2b.reshape(1, D))
    return jnp.concatenate([x, h1, h2], axis=1)


# speedup vs baseline: 1.1287x; 1.1287x over previous
import functools

import jax
import jax.numpy as jnp
from jax import lax
from jax.experimental import pallas as pl
from jax.experimental.pallas import tpu as pltpu
from jax.experimental.pallas import tpu_sc as plsc

N = 10000
E = 320000
D = 128
DC = 64

NC = 2
NS = 16
K = 128
C = 160
PHASES = 4
CP = C // PHASES
NSLOT = 2
T = CP // NSLOT
EPT = K * C
E_PAD = NS * EPT
NPAD = 10112
RPT = NPAD // NS
DUMMY = N + 8
_RCHUNKS = [128, 128, 128, 128, 120]
_XROWS = 624
_XCHUNKS = [128, 128, 128, 128, 112]

_mesh = plsc.VectorSubcoreMesh(core_axis_name="c", subcore_axis_name="s")


@functools.partial(
    pl.kernel,
    out_type=jax.ShapeDtypeStruct((NC, NS, RPT, DC), jnp.float32),
    mesh=_mesh,
    scratch_types=[
        pltpu.VMEM_SHARED((N, DC), jnp.float32),
        pltpu.VMEM_SHARED((NPAD, DC), jnp.float32),
        pltpu.VMEM((CP * K,), jnp.int32),
        [pltpu.VMEM((1, K), jnp.int32) for _ in range(NSLOT)],
        [pltpu.VMEM((K, DC), jnp.float32) for _ in range(NSLOT)],
        [pltpu.SemaphoreType.DMA for _ in range(3 * NSLOT)],
    ],
)
def _segment_sum_sc(xt_hbm, src_hbm, dst_hbm, zero_hbm, out_hbm,
                    xsp, acc, src_v, dbufs, bufs, sems):
    gsems = sems[0:NSLOT]
    dsems = sems[NSLOT:2 * NSLOT]
    ssems = sems[2 * NSLOT:3 * NSLOT]
    c = lax.axis_index("c")
    s = lax.axis_index("s")

    xb = s * _XROWS
    off = 0
    for i, sz in enumerate(_XCHUNKS):
        b = bufs[i % NSLOT]
        pltpu.sync_copy(xt_hbm.at[c, pl.ds(xb + off, sz)], b.at[pl.ds(0, sz)])
        pltpu.sync_copy(b.at[pl.ds(0, sz)], xsp.at[pl.ds(xb + off, sz)])
        off += sz

    @pl.when(s == 0)
    def _():
        xtail = NS * _XROWS
        pltpu.sync_copy(xt_hbm.at[c, pl.ds(xtail, N - xtail)],
                        bufs[0].at[pl.ds(0, N - xtail)])
        pltpu.sync_copy(bufs[0].at[pl.ds(0, N - xtail)],
                        xsp.at[pl.ds(xtail, N - xtail)])

    r0 = s * RPT
    pltpu.sync_copy(zero_hbm, bufs[1])
    off = 0
    for i, sz in enumerate(_RCHUNKS):
        pltpu.sync_copy(bufs[1].at[pl.ds(0, sz)], acc.at[pl.ds(r0 + off, sz)])
        off += sz

    plsc.subcore_barrier()

    def start_in(pbase, j, i):
        pltpu.async_copy(xsp.at[src_v.at[pl.ds(j * K, K)]], bufs[i], gsems[i])
        pltpu.async_copy(dst_hbm.at[pl.ds(s * C + pbase + j, 1)],
                         dbufs[i], dsems[i])

    def wait_in(i):
        pltpu.make_async_copy(xsp.at[src_v.at[pl.ds(0, K)]],
                              bufs[i], gsems[i]).wait()
        pltpu.make_async_copy(dst_hbm.at[pl.ds(0, 1)], dbufs[i],
                              dsems[i]).wait()

    def start_scat(i):
        pltpu.async_copy(bufs[i], acc.at[dbufs[i].at[0]], ssems[i], add=True)

    def wait_scat(i):
        pltpu.make_async_copy(bufs[i], acc.at[dbufs[i].at[0]],
                              ssems[i]).wait()

    for p in range(PHASES):
        pbase = p * CP
        pltpu.sync_copy(src_hbm.at[pl.ds(s * EPT + pbase * K, CP * K)], src_v)
        for i in range(NSLOT):
            start_in(pbase, i, i)

        def body(t, carry, pbase=pbase):
            j0 = NSLOT * t
            for i in range(NSLOT):
                wait_in(i)
                start_scat(i)
            for i in range(NSLOT):
                wait_scat(i)

                @pl.when(t + 1 < T)
                def _(i=i):
                    start_in(pbase, j0 + NSLOT + i, i)
            return carry

        lax.fori_loop(0, T, body, 0)
    plsc.subcore_barrier()

    off = 0
    for i, sz in enumerate(_RCHUNKS):
        b = bufs[i % NSLOT]
        pltpu.sync_copy(acc.at[pl.ds(r0 + off, sz)], b.at[pl.ds(0, sz)])
        pltpu.sync_copy(b.at[pl.ds(0, sz)], out_hbm.at[c, s, pl.ds(off, sz)])
        off += sz


def _mlp_body(x_ref, p0_ref, p1_ref, w1_ref, b1_ref, w2_ref, b2_ref, o_ref):
    agg = jnp.concatenate([p0_ref[0], p1_ref[0]], axis=1)
    sm = x_ref[...] + agg
    h = jnp.dot(sm, w1_ref[...], preferred_element_type=jnp.float32)
    h = jnp.maximum(h + b1_ref[...], 0.0)
    o = jnp.dot(h, w2_ref[...], preferred_element_type=jnp.float32)
    o_ref[...] = jnp.maximum(o + b2_ref[...], 0.0)


_BLK = 1000


def _mlp_tc(x, parts, W1, b1, W2, b2):
    grid = (N // _BLK,)
    return pl.pallas_call(
        _mlp_body,
        grid=grid,
        in_specs=[
            pl.BlockSpec((_BLK, D), lambda i: (i, 0)),
            pl.BlockSpec((1, _BLK, DC), lambda i: (0, i, 0)),
            pl.BlockSpec((1, _BLK, DC), lambda i: (1, i, 0)),
            pl.BlockSpec((D, D), lambda i: (0, 0)),
            pl.BlockSpec((1, D), lambda i: (0, 0)),
            pl.BlockSpec((D, D), lambda i: (0, 0)),
            pl.BlockSpec((1, D), lambda i: (0, 0)),
        ],
        out_specs=pl.BlockSpec((_BLK, D), lambda i: (i, 0)),
        out_shape=jax.ShapeDtypeStruct((N, D), jnp.float32),
    )(x, parts, parts, W1, b1, W2, b2)


def kernel(x, edge_index, W1a, b1a, W1b, b1b, W2a, b2a, W2b, b2b):
    pad = E_PAD - E
    src = jnp.concatenate([edge_index[0], jnp.zeros((pad,), jnp.int32)])
    dst = jnp.concatenate([edge_index[1], jnp.full((pad,), DUMMY, jnp.int32)])
    dst = dst.reshape(NS * C, K)
    zero = jnp.zeros((K, DC), jnp.float32)

    xt = jnp.stack([x[:, :DC], x[:, DC:]], axis=0)
    parts1 = _segment_sum_sc(xt, src, dst, zero).reshape(NC, NPAD, DC)
    h1 = _mlp_tc(x, parts1, W1a, b1a.reshape(1, D), W1b, b1b.reshape(1, D))
    h1t = jnp.stack([h1[:, :DC], h1[:, DC:]], axis=0)
    parts2 = _segment_sum_sc(h1t, src, dst, zero).reshape(NC, NPAD, DC)
    h2 = _mlp_tc(h1, parts2, W2a, b2a.reshape(1, D), W2b, b2b.reshape(1, D))
    return jnp.concatenate([x, h1, h2], axis=1)


# speedup vs baseline: 1.2694x; 1.1247x over previous
import functools

import jax
import jax.numpy as jnp
from jax import lax
from jax.experimental import pallas as pl
from jax.experimental.pallas import tpu as pltpu
from jax.experimental.pallas import tpu_sc as plsc

N = 10000
E = 320000
D = 128
DC = 64

NC = 2
NS = 16
K = 128
C = 160
PHASES = 2
CP = C // PHASES
NSLOT = 2
T = CP // NSLOT
EPT = K * C
E_PAD = NS * EPT
NPAD = 10112
RPT = NPAD // NS
DUMMY = N + 8
_RCHUNKS = [128, 128, 128, 128, 120]
_XROWS = 624
_XCHUNKS = [128, 128, 128, 128, 112]

_mesh = plsc.VectorSubcoreMesh(core_axis_name="c", subcore_axis_name="s")


@functools.partial(
    pl.kernel,
    out_type=jax.ShapeDtypeStruct((NC, NS, RPT, DC), jnp.float32),
    mesh=_mesh,
    scratch_types=[
        pltpu.VMEM_SHARED((N, DC), jnp.float32),
        pltpu.VMEM_SHARED((NPAD, DC), jnp.float32),
        pltpu.VMEM((CP * K,), jnp.int32),
        [pltpu.VMEM((1, K), jnp.int32) for _ in range(NSLOT)],
        [pltpu.VMEM((K, DC), jnp.float32) for _ in range(NSLOT)],
        [pltpu.SemaphoreType.DMA for _ in range(2 * NSLOT)],
    ],
)
def _segment_sum_sc(xt_hbm, src_hbm, dst_hbm, zero_hbm, out_hbm,
                    xsp, acc, src_v, dbufs, bufs, sems):
    gsems = sems[0:NSLOT]
    dsems = sems[NSLOT:2 * NSLOT]
    c = lax.axis_index("c")
    s = lax.axis_index("s")

    xb = s * _XROWS
    off = 0
    for i, sz in enumerate(_XCHUNKS):
        b = bufs[i % NSLOT]
        pltpu.sync_copy(xt_hbm.at[c, pl.ds(xb + off, sz)], b.at[pl.ds(0, sz)])
        pltpu.sync_copy(b.at[pl.ds(0, sz)], xsp.at[pl.ds(xb + off, sz)])
        off += sz

    @pl.when(s == 0)
    def _():
        xtail = NS * _XROWS
        pltpu.sync_copy(xt_hbm.at[c, pl.ds(xtail, N - xtail)],
                        bufs[0].at[pl.ds(0, N - xtail)])
        pltpu.sync_copy(bufs[0].at[pl.ds(0, N - xtail)],
                        xsp.at[pl.ds(xtail, N - xtail)])

    r0 = s * RPT
    pltpu.sync_copy(zero_hbm, bufs[1])
    off = 0
    for i, sz in enumerate(_RCHUNKS):
        pltpu.sync_copy(bufs[1].at[pl.ds(0, sz)], acc.at[pl.ds(r0 + off, sz)])
        off += sz

    plsc.subcore_barrier()

    def start_in(pbase, j, i):
        pltpu.async_copy(xsp.at[src_v.at[pl.ds(j * K, K)]], bufs[i], gsems[i])
        pltpu.async_copy(dst_hbm.at[pl.ds(s * C + pbase + j, 1)],
                         dbufs[i], dsems[i])

    def wait_in(i):
        pltpu.make_async_copy(xsp.at[src_v.at[pl.ds(0, K)]],
                              bufs[i], gsems[i]).wait()
        pltpu.make_async_copy(dst_hbm.at[pl.ds(0, 1)], dbufs[i],
                              dsems[i]).wait()

    def scat(i):
        pltpu.sync_copy(bufs[i], acc.at[dbufs[i].at[0]], add=True)

    for p in range(PHASES):
        pbase = p * CP
        pltpu.sync_copy(src_hbm.at[pl.ds(s * EPT + pbase * K, CP * K)], src_v)
        start_in(pbase, 0, 0)

        def body(t, carry, pbase=pbase):
            j0 = 2 * t
            j1 = j0 + 1
            start_in(pbase, j1, 1)
            wait_in(0)
            scat(0)

            @pl.when(j1 + 1 < CP)
            def _():
                start_in(pbase, j1 + 1, 0)

            wait_in(1)
            scat(1)
            return carry

        lax.fori_loop(0, CP // 2, body, 0)
    plsc.subcore_barrier()

    off = 0
    for i, sz in enumerate(_RCHUNKS):
        b = bufs[i % NSLOT]
        pltpu.sync_copy(acc.at[pl.ds(r0 + off, sz)], b.at[pl.ds(0, sz)])
        pltpu.sync_copy(b.at[pl.ds(0, sz)], out_hbm.at[c, s, pl.ds(off, sz)])
        off += sz


def _mlp1_body(x_ref, p0_ref, p1_ref, w1_ref, b1_ref, w2_ref, b2_ref, o_ref):
    agg = jnp.concatenate([p0_ref[0], p1_ref[0]], axis=1)
    sm = x_ref[...] + agg
    h = jnp.dot(sm, w1_ref[...], preferred_element_type=jnp.float32)
    h = jnp.maximum(h + b1_ref[...], 0.0)
    o = jnp.dot(h, w2_ref[...], preferred_element_type=jnp.float32)
    o = jnp.maximum(o + b2_ref[...], 0.0)
    o_ref[0] = o[:, :DC]
    o_ref[1] = o[:, DC:]


def _mlp2_body(x_ref, h0_ref, h1_ref, p0_ref, p1_ref,
               w1_ref, b1_ref, w2_ref, b2_ref, o_ref):
    h1full = jnp.concatenate([h0_ref[0], h1_ref[0]], axis=1)
    agg = jnp.concatenate([p0_ref[0], p1_ref[0]], axis=1)
    sm = h1full + agg
    h = jnp.dot(sm, w1_ref[...], preferred_element_type=jnp.float32)
    h = jnp.maximum(h + b1_ref[...], 0.0)
    o = jnp.dot(h, w2_ref[...], preferred_element_type=jnp.float32)
    o = jnp.maximum(o + b2_ref[...], 0.0)
    o_ref[:, 0:D] = x_ref[...]
    o_ref[:, D:2 * D] = h1full
    o_ref[:, 2 * D:] = o


_BLK = 1000
_WSPECS = [
    pl.BlockSpec((D, D), lambda i: (0, 0)),
    pl.BlockSpec((1, D), lambda i: (0, 0)),
    pl.BlockSpec((D, D), lambda i: (0, 0)),
    pl.BlockSpec((1, D), lambda i: (0, 0)),
]
_PSPECS = [
    pl.BlockSpec((1, _BLK, DC), lambda i: (0, i, 0)),
    pl.BlockSpec((1, _BLK, DC), lambda i: (1, i, 0)),
]


def _mlp1_tc(x, parts, W1, b1, W2, b2):
    return pl.pallas_call(
        _mlp1_body,
        grid=(N // _BLK,),
        in_specs=[pl.BlockSpec((_BLK, D), lambda i: (i, 0))] + _PSPECS + _WSPECS,
        out_specs=pl.BlockSpec((NC, _BLK, DC), lambda i: (0, i, 0)),
        out_shape=jax.ShapeDtypeStruct((NC, N, DC), jnp.float32),
    )(x, parts, parts, W1, b1, W2, b2)


def _mlp2_tc(x, h1t, parts, W1, b1, W2, b2):
    return pl.pallas_call(
        _mlp2_body,
        grid=(N // _BLK,),
        in_specs=([pl.BlockSpec((_BLK, D), lambda i: (i, 0))]
                  + _PSPECS + _PSPECS + _WSPECS),
        out_specs=pl.BlockSpec((_BLK, 3 * D), lambda i: (i, 0)),
        out_shape=jax.ShapeDtypeStruct((N, 3 * D), jnp.float32),
    )(x, h1t, h1t, parts, parts, W1, b1, W2, b2)


def kernel(x, edge_index, W1a, b1a, W1b, b1b, W2a, b2a, W2b, b2b):
    pad = E_PAD - E
    src = jnp.concatenate([edge_index[0], jnp.zeros((pad,), jnp.int32)])
    dst = jnp.concatenate([edge_index[1], jnp.full((pad,), DUMMY, jnp.int32)])
    dst = dst.reshape(NS * C, K)
    zero = jnp.zeros((K, DC), jnp.float32)

    xt = jnp.stack([x[:, :DC], x[:, DC:]], axis=0)
    parts1 = _segment_sum_sc(xt, src, dst, zero).reshape(NC, NPAD, DC)
    h1t = _mlp1_tc(x, parts1, W1a, b1a.reshape(1, D), W1b, b1b.reshape(1, D))
    parts2 = _segment_sum_sc(h1t, src, dst, zero).reshape(NC, NPAD, DC)
    return _mlp2_tc(x, h1t, parts2,
                    W2a, b2a.reshape(1, D), W2b, b2b.reshape(1, D))


# speedup vs baseline: 1.3040x; 1.0272x over previous
import functools

import jax
import jax.numpy as jnp
from jax import lax
from jax.experimental import pallas as pl
from jax.experimental.pallas import tpu as pltpu
from jax.experimental.pallas import tpu_sc as plsc

N = 10000
E = 320000
D = 128
DC = 64

NC = 2
NS = 16
K = 128
C = 160
PHASES = 2
CP = C // PHASES
NSLOT = 2
T = CP // NSLOT
EPT = K * C
E_PAD = NS * EPT
NPAD = 10112
RPT = NPAD // NS
DUMMY = N + 8
_RCHUNKS = [128, 128, 128, 128, 120]
_XROWS = 624
_XCHUNKS = [128, 128, 128, 128, 112]

_mesh = plsc.VectorSubcoreMesh(core_axis_name="c", subcore_axis_name="s")


@functools.partial(
    pl.kernel,
    out_type=jax.ShapeDtypeStruct((NC, NS, RPT, DC), jnp.float32),
    mesh=_mesh,
    scratch_types=[
        pltpu.VMEM_SHARED((N, DC), jnp.float32),
        pltpu.VMEM_SHARED((NPAD, DC), jnp.float32),
        pltpu.VMEM((CP * K,), jnp.int32),
        [pltpu.VMEM((1, K), jnp.int32) for _ in range(NSLOT)],
        [pltpu.VMEM((K, DC), jnp.float32) for _ in range(NSLOT)],
        [pltpu.SemaphoreType.DMA for _ in range(2 * NSLOT)],
    ],
)
def _segment_sum_sc(xt_hbm, src_hbm, dst_hbm, zero_hbm, out_hbm,
                    xsp, acc, src_v, dbufs, bufs, sems):
    gsems = sems[0:NSLOT]
    dsems = sems[NSLOT:2 * NSLOT]
    c = lax.axis_index("c")
    s = lax.axis_index("s")

    xb = s * _XROWS
    pltpu.sync_copy(xt_hbm.at[c, pl.ds(xb, _XROWS)], xsp.at[pl.ds(xb, _XROWS)])

    @pl.when(s == 0)
    def _():
        xtail = NS * _XROWS
        pltpu.sync_copy(xt_hbm.at[c, pl.ds(xtail, N - xtail)],
                        xsp.at[pl.ds(xtail, N - xtail)])

    r0 = s * RPT
    pltpu.sync_copy(zero_hbm, bufs[1])
    off = 0
    for i, sz in enumerate(_RCHUNKS):
        pltpu.sync_copy(bufs[1].at[pl.ds(0, sz)], acc.at[pl.ds(r0 + off, sz)])
        off += sz

    plsc.subcore_barrier()

    def start_in(pbase, j, i):
        pltpu.async_copy(xsp.at[src_v.at[pl.ds(j * K, K)]], bufs[i], gsems[i])
        pltpu.async_copy(dst_hbm.at[pl.ds(s * C + pbase + j, 1)],
                         dbufs[i], dsems[i])

    def wait_in(i):
        pltpu.make_async_copy(xsp.at[src_v.at[pl.ds(0, K)]],
                              bufs[i], gsems[i]).wait()
        pltpu.make_async_copy(dst_hbm.at[pl.ds(0, 1)], dbufs[i],
                              dsems[i]).wait()

    def scat(i):
        pltpu.sync_copy(bufs[i], acc.at[dbufs[i].at[0]], add=True)

    for p in range(PHASES):
        pbase = p * CP
        pltpu.sync_copy(src_hbm.at[pl.ds(s * EPT + pbase * K, CP * K)], src_v)
        start_in(pbase, 0, 0)

        def body(t, carry, pbase=pbase):
            j0 = 2 * t
            j1 = j0 + 1
            start_in(pbase, j1, 1)
            wait_in(0)
            scat(0)

            @pl.when(j1 + 1 < CP)
            def _():
                start_in(pbase, j1 + 1, 0)

            wait_in(1)
            scat(1)
            return carry

        lax.fori_loop(0, CP // 2, body, 0)
    plsc.subcore_barrier()

    pltpu.sync_copy(acc.at[pl.ds(r0, RPT)], out_hbm.at[c, s])


def _mlp1_body(x_ref, p0_ref, p1_ref, w1_ref, b1_ref, w2_ref, b2_ref, o_ref):
    agg = jnp.concatenate([p0_ref[0], p1_ref[0]], axis=1)
    sm = x_ref[...] + agg
    h = jnp.dot(sm, w1_ref[...], preferred_element_type=jnp.float32)
    h = jnp.maximum(h + b1_ref[...], 0.0)
    o = jnp.dot(h, w2_ref[...], preferred_element_type=jnp.float32)
    o = jnp.maximum(o + b2_ref[...], 0.0)
    o_ref[0] = o[:, :DC]
    o_ref[1] = o[:, DC:]


def _mlp2_body(x_ref, h0_ref, h1_ref, p0_ref, p1_ref,
               w1_ref, b1_ref, w2_ref, b2_ref, o_ref):
    h1full = jnp.concatenate([h0_ref[0], h1_ref[0]], axis=1)
    agg = jnp.concatenate([p0_ref[0], p1_ref[0]], axis=1)
    sm = h1full + agg
    h = jnp.dot(sm, w1_ref[...], preferred_element_type=jnp.float32)
    h = jnp.maximum(h + b1_ref[...], 0.0)
    o = jnp.dot(h, w2_ref[...], preferred_element_type=jnp.float32)
    o = jnp.maximum(o + b2_ref[...], 0.0)
    o_ref[:, 0:D] = x_ref[...]
    o_ref[:, D:2 * D] = h1full
    o_ref[:, 2 * D:] = o


_BLK = 1000
_WSPECS = [
    pl.BlockSpec((D, D), lambda i: (0, 0)),
    pl.BlockSpec((1, D), lambda i: (0, 0)),
    pl.BlockSpec((D, D), lambda i: (0, 0)),
    pl.BlockSpec((1, D), lambda i: (0, 0)),
]
_PSPECS = [
    pl.BlockSpec((1, _BLK, DC), lambda i: (0, i, 0)),
    pl.BlockSpec((1, _BLK, DC), lambda i: (1, i, 0)),
]


def _mlp1_tc(x, parts, W1, b1, W2, b2):
    return pl.pallas_call(
        _mlp1_body,
        grid=(N // _BLK,),
        in_specs=[pl.BlockSpec((_BLK, D), lambda i: (i, 0))] + _PSPECS + _WSPECS,
        out_specs=pl.BlockSpec((NC, _BLK, DC), lambda i: (0, i, 0)),
        out_shape=jax.ShapeDtypeStruct((NC, N, DC), jnp.float32),
    )(x, parts, parts, W1, b1, W2, b2)


def _mlp2_tc(x, h1t, parts, W1, b1, W2, b2):
    return pl.pallas_call(
        _mlp2_body,
        grid=(N // _BLK,),
        in_specs=([pl.BlockSpec((_BLK, D), lambda i: (i, 0))]
                  + _PSPECS + _PSPECS + _WSPECS),
        out_specs=pl.BlockSpec((_BLK, 3 * D), lambda i: (i, 0)),
        out_shape=jax.ShapeDtypeStruct((N, 3 * D), jnp.float32),
    )(x, h1t, h1t, parts, parts, W1, b1, W2, b2)


def kernel(x, edge_index, W1a, b1a, W1b, b1b, W2a, b2a, W2b, b2b):
    pad = E_PAD - E
    src = jnp.concatenate([edge_index[0], jnp.zeros((pad,), jnp.int32)])
    dst = jnp.concatenate([edge_index[1], jnp.full((pad,), DUMMY, jnp.int32)])
    dst = dst.reshape(NS * C, K)
    zero = jnp.zeros((K, DC), jnp.float32)

    xt = jnp.stack([x[:, :DC], x[:, DC:]], axis=0)
    parts1 = _segment_sum_sc(xt, src, dst, zero).reshape(NC, NPAD, DC)
    h1t = _mlp1_tc(x, parts1, W1a, b1a.reshape(1, D), W1b, b1b.reshape(1, D))
    parts2 = _segment_sum_sc(h1t, src, dst, zero).reshape(NC, NPAD, DC)
    return _mlp2_tc(x, h1t, parts2,
                    W2a, b2a.reshape(1, D), W2b, b2b.reshape(1, D))


# speedup vs baseline: 1.3247x; 1.0158x over previous
import functools

import jax
import jax.numpy as jnp
from jax import lax
from jax.experimental import pallas as pl
from jax.experimental.pallas import tpu as pltpu
from jax.experimental.pallas import tpu_sc as plsc

N = 10000
E = 320000
D = 128
DC = 64

NC = 2
NS = 16
K = 128
C = 160
PHASES = 2
CP = C // PHASES
NSLOT = 2
EPT = K * C
E_PAD = NS * EPT
NPAD = 10112
RPT = NPAD // NS
DUMMY = N + 8
_RCHUNKS = [128, 128, 128, 128, 120]
_XROWS = 624

_mesh = plsc.VectorSubcoreMesh(core_axis_name="c", subcore_axis_name="s")


@functools.partial(
    pl.kernel,
    out_type=jax.ShapeDtypeStruct((NC, NS, RPT, DC), jnp.float32),
    mesh=_mesh,
    scratch_types=[
        pltpu.VMEM_SHARED((N, DC), jnp.float32),
        pltpu.VMEM_SHARED((NPAD, DC), jnp.float32),
        pltpu.VMEM((CP * K,), jnp.int32),
        [pltpu.VMEM((1, K), jnp.int32) for _ in range(NSLOT)],
        [pltpu.VMEM((K, DC), jnp.float32) for _ in range(NSLOT)],
        [pltpu.SemaphoreType.DMA for _ in range(2 * NSLOT)],
    ],
)
def _segment_sum_sc(xt_hbm, src_hbm, dst_hbm, zero_hbm, out_hbm,
                    xsp, acc, src_v, dbufs, bufs, sems):
    gsems = sems[0:NSLOT]
    dsems = sems[NSLOT:2 * NSLOT]
    c = lax.axis_index("c")
    s = lax.axis_index("s")

    xb = s * _XROWS
    pltpu.sync_copy(xt_hbm.at[c, pl.ds(xb, _XROWS)], xsp.at[pl.ds(xb, _XROWS)])

    @pl.when(s == 0)
    def _():
        xtail = NS * _XROWS
        pltpu.sync_copy(xt_hbm.at[c, pl.ds(xtail, N - xtail)],
                        xsp.at[pl.ds(xtail, N - xtail)])

    r0 = s * RPT
    pltpu.sync_copy(zero_hbm, bufs[1])
    off = 0
    for i, sz in enumerate(_RCHUNKS):
        pltpu.sync_copy(bufs[1].at[pl.ds(0, sz)], acc.at[pl.ds(r0 + off, sz)])
        off += sz

    plsc.subcore_barrier()

    def start_in(pbase, j, i):
        pltpu.async_copy(xsp.at[src_v.at[pl.ds(j * K, K)]], bufs[i], gsems[i])
        pltpu.async_copy(dst_hbm.at[pl.ds(s * C + pbase + j, 1)],
                         dbufs[i], dsems[i])

    def wait_in(i):
        pltpu.make_async_copy(xsp.at[src_v.at[pl.ds(0, K)]],
                              bufs[i], gsems[i]).wait()
        pltpu.make_async_copy(dst_hbm.at[pl.ds(0, 1)], dbufs[i],
                              dsems[i]).wait()

    def scat(i):
        pltpu.sync_copy(bufs[i], acc.at[dbufs[i].at[0]], add=True)

    for p in range(PHASES):
        pbase = p * CP
        pltpu.sync_copy(src_hbm.at[pl.ds(s * EPT + pbase * K, CP * K)], src_v)
        start_in(pbase, 0, 0)

        def body(t, carry, pbase=pbase):
            j0 = 2 * t
            j1 = j0 + 1
            start_in(pbase, j1, 1)
            wait_in(0)
            scat(0)

            @pl.when(j1 + 1 < CP)
            def _():
                start_in(pbase, j1 + 1, 0)

            wait_in(1)
            scat(1)
            return carry

        lax.fori_loop(0, CP // 2, body, 0)
    plsc.subcore_barrier()

    pltpu.sync_copy(acc.at[pl.ds(r0, RPT)], out_hbm.at[c, s])


def _mlp1_body(x_ref, p0_ref, p1_ref, w1_ref, b1_ref, w2_ref, b2_ref, o_ref):
    agg = jnp.concatenate([p0_ref[0], p1_ref[0]], axis=1)
    sm = x_ref[...] + agg
    h = jnp.dot(sm, w1_ref[...], preferred_element_type=jnp.float32)
    h = jnp.maximum(h + b1_ref[...], 0.0)
    o = jnp.dot(h, w2_ref[...], preferred_element_type=jnp.float32)
    o = jnp.maximum(o + b2_ref[...], 0.0)
    o_ref[0] = o[:, :DC]
    o_ref[1] = o[:, DC:]


def _mlp2_body(x_ref, h0_ref, h1_ref, p0_ref, p1_ref,
               w1_ref, b1_ref, w2_ref, b2_ref, o_ref):
    h1full = jnp.concatenate([h0_ref[0], h1_ref[0]], axis=1)
    agg = jnp.concatenate([p0_ref[0], p1_ref[0]], axis=1)
    sm = h1full + agg
    h = jnp.dot(sm, w1_ref[...], preferred_element_type=jnp.float32)
    h = jnp.maximum(h + b1_ref[...], 0.0)
    o = jnp.dot(h, w2_ref[...], preferred_element_type=jnp.float32)
    o = jnp.maximum(o + b2_ref[...], 0.0)
    o_ref[:, 0:D] = x_ref[...]
    o_ref[:, D:2 * D] = h1full
    o_ref[:, 2 * D:] = o


_BLK = 2000
_WSPECS = [
    pl.BlockSpec((D, D), lambda i: (0, 0)),
    pl.BlockSpec((1, D), lambda i: (0, 0)),
    pl.BlockSpec((D, D), lambda i: (0, 0)),
    pl.BlockSpec((1, D), lambda i: (0, 0)),
]
_PSPECS = [
    pl.BlockSpec((1, _BLK, DC), lambda i: (0, i, 0)),
    pl.BlockSpec((1, _BLK, DC), lambda i: (1, i, 0)),
]


def _mlp1_tc(x, parts, W1, b1, W2, b2):
    return pl.pallas_call(
        _mlp1_body,
        grid=(N // _BLK,),
        in_specs=[pl.BlockSpec((_BLK, D), lambda i: (i, 0))] + _PSPECS + _WSPECS,
        out_specs=pl.BlockSpec((NC, _BLK, DC), lambda i: (0, i, 0)),
        out_shape=jax.ShapeDtypeStruct((NC, N, DC), jnp.float32),
    )(x, parts, parts, W1, b1, W2, b2)


def _mlp2_tc(x, h1t, parts, W1, b1, W2, b2):
    return pl.pallas_call(
        _mlp2_body,
        grid=(N // _BLK,),
        in_specs=([pl.BlockSpec((_BLK, D), lambda i: (i, 0))]
                  + _PSPECS + _PSPECS + _WSPECS),
        out_specs=pl.BlockSpec((_BLK, 3 * D), lambda i: (i, 0)),
        out_shape=jax.ShapeDtypeStruct((N, 3 * D), jnp.float32),
    )(x, h1t, h1t, parts, parts, W1, b1, W2, b2)


def kernel(x, edge_index, W1a, b1a, W1b, b1b, W2a, b2a, W2b, b2b):
    pad = E_PAD - E
    src = jnp.concatenate([edge_index[0], jnp.zeros((pad,), jnp.int32)])
    dst = jnp.concatenate([edge_index[1], jnp.full((pad,), DUMMY, jnp.int32)])
    dst = dst.reshape(NS * C, K)
    zero = jnp.zeros((K, DC), jnp.float32)

    xt = jnp.stack([x[:, :DC], x[:, DC:]], axis=0)
    parts1 = _segment_sum_sc(xt, src, dst, zero).reshape(NC, NPAD, DC)
    h1t = _mlp1_tc(x, parts1, W1a, b1a.reshape(1, D), W1b, b1b.reshape(1, D))
    parts2 = _segment_sum_sc(h1t, src, dst, zero).reshape(NC, NPAD, DC)
    return _mlp2_tc(x, h1t, parts2,
                    W2a, b2a.reshape(1, D), W2b, b2b.reshape(1, D))
